# Initial kernel scaffold; baseline (speedup 1.0000x reference)
import jax
import jax.numpy as jnp
from jax.experimental import pallas as pl


def kernel(A_x, B_x, A_edge_index, B_edge_index, Wl1, bl1, Wr1, Wl2, bl2, Wr2, Pw1, Pb1, Pw2, Pb2):
    raise NotImplementedError("write your pallas kernel here")


# speedup vs baseline: 4.1036x; 4.1036x over previous
import functools

import jax
import jax.numpy as jnp
from jax import lax
from jax.experimental import pallas as pl
from jax.experimental.pallas import tpu as pltpu
from jax.experimental.pallas import tpu_sc as plsc

N_NODES = 10000
N_EDGES = 320000
DIM = 128
HALF = DIM // 2
PROJ = 64
NUM_CORES = 2
NUM_SUBCORES = 16
CHUNK = 128
NCH = N_EDGES // CHUNK
CH_PER_TILE = NCH // NUM_SUBCORES
CH_EXTRA = NCH - CH_PER_TILE * NUM_SUBCORES
CH_BUF = CH_PER_TILE + 1
RPT = 632
RPT_LAST = N_NODES - (NUM_SUBCORES - 1) * RPT
ROW_BLK = 2000


def _make_segsum(with_cnt: bool):
    mesh = plsc.VectorSubcoreMesh(core_axis_name="c", subcore_axis_name="s",
                                  num_cores=NUM_CORES, num_subcores=NUM_SUBCORES)
    out_type = [jax.ShapeDtypeStruct((NUM_CORES, 2, N_NODES, HALF),
                                     jnp.float32)]
    if with_cnt:
        out_type.append(jax.ShapeDtypeStruct((NUM_CORES, N_NODES), jnp.float32))
    scratch = [
        pltpu.VMEM((CH_BUF, CHUNK), jnp.int32),
        pltpu.VMEM((CH_BUF, CHUNK), jnp.int32),
        pltpu.VMEM((CHUNK, HALF), jnp.float32),
        pltpu.VMEM((CHUNK,), jnp.float32),
        pltpu.VMEM_SHARED((N_NODES, HALF), jnp.float32),
        pltpu.VMEM_SHARED((N_NODES,), jnp.float32),
        pltpu.SemaphoreType.DMA,
    ]

    def body(x_hbm, src_hbm, dst_hbm, zrows_hbm, zcnt_hbm, agg_hbm, *rest):
        if with_cnt:
            cnt_hbm = rest[0]
            rest = rest[1:]
        src_v, dst_v, rows_v, ones_v, acc_sh, cnt_sh, sem = rest
        core = lax.axis_index("c")
        sub = lax.axis_index("s")

        def zero_acc():
            @pl.when(sub < NUM_SUBCORES - 1)
            def _zero_full():
                pltpu.sync_copy(zrows_hbm, acc_sh.at[pl.ds(sub * RPT, RPT)])

            @pl.when(sub == NUM_SUBCORES - 1)
            def _zero_last():
                pltpu.sync_copy(
                    zrows_hbm.at[pl.ds(0, RPT_LAST)],
                    acc_sh.at[pl.ds((NUM_SUBCORES - 1) * RPT, RPT_LAST)])

        def write_acc(half):
            @pl.when(sub < NUM_SUBCORES - 1)
            def _out_full():
                pltpu.sync_copy(acc_sh.at[pl.ds(sub * RPT, RPT)],
                                agg_hbm.at[core].at[half]
                                .at[pl.ds(sub * RPT, RPT)])

            @pl.when(sub == NUM_SUBCORES - 1)
            def _out_last():
                pltpu.sync_copy(
                    acc_sh.at[pl.ds((NUM_SUBCORES - 1) * RPT, RPT_LAST)],
                    agg_hbm.at[core].at[half]
                    .at[pl.ds((NUM_SUBCORES - 1) * RPT, RPT_LAST)])

        if with_cnt:
            @pl.when(sub == 0)
            def _zero_cnt():
                pltpu.sync_copy(zcnt_hbm, cnt_sh)
            for i in range(CHUNK // 16):
                ones_v[pl.ds(i * 16, 16)] = jnp.ones((16,), jnp.float32)

        pltpu.sync_copy(src_hbm.at[core].at[sub], src_v)
        pltpu.sync_copy(dst_hbm.at[core].at[sub], dst_v)

        nch = CH_PER_TILE + jnp.where(sub < CH_EXTRA, 1, 0)
        zero_acc()
        plsc.subcore_barrier()

        for half in range(2):
            def step(j, carry):
                pltpu.async_copy(x_hbm.at[core].at[half].at[src_v.at[j]],
                                 rows_v, sem).wait()
                pltpu.sync_copy(rows_v, acc_sh.at[dst_v.at[j]], add=True)
                if with_cnt and half == 0:
                    pltpu.sync_copy(ones_v, cnt_sh.at[dst_v.at[j]], add=True)
                return carry

            lax.fori_loop(0, nch, step, 0)
            plsc.subcore_barrier()
            write_acc(half)
            if half == 0:
                if with_cnt:
                    @pl.when(sub == 0)
                    def _out_cnt():
                        pltpu.sync_copy(cnt_sh, cnt_hbm.at[core])
                zero_acc()
                plsc.subcore_barrier()

    return pl.kernel(body, out_type=tuple(out_type), mesh=mesh,
                     scratch_types=scratch,
                     compiler_params=pltpu.CompilerParams(
                         use_tc_tiling_on_sc=False))


_make_segsum = functools.lru_cache(maxsize=None)(_make_segsum)


def _dense1_body(agg_ref, cnt_ref, x_ref, wl_ref, bl_ref, wr_ref, h_ref):
    a = agg_ref[0] / jnp.maximum(cnt_ref[0], 1.0)
    h = (jnp.dot(a, wl_ref[...], preferred_element_type=jnp.float32)
         + bl_ref[...]
         + jnp.dot(x_ref[0], wr_ref[...], preferred_element_type=jnp.float32))
    h_ref[0] = jnp.maximum(h, 0.0)


def _dense2_body(agg_ref, cnt_ref, x_ref, wl_ref, bl_ref, wr_ref,
                 pw1_ref, pb1_ref, pw2_ref, pb2_ref, h_ref, z_ref):
    a = agg_ref[0] / jnp.maximum(cnt_ref[0], 1.0)
    h = (jnp.dot(a, wl_ref[...], preferred_element_type=jnp.float32)
         + bl_ref[...]
         + jnp.dot(x_ref[0], wr_ref[...], preferred_element_type=jnp.float32))
    h_ref[0] = h
    t = jnp.maximum(
        jnp.dot(h, pw1_ref[...], preferred_element_type=jnp.float32)
        + pb1_ref[...], 0.0)
    z = (jnp.dot(t, pw2_ref[...], preferred_element_type=jnp.float32)
         + pb2_ref[...])
    nrm = jnp.sqrt(jnp.sum(z * z, axis=1, keepdims=True))
    z_ref[0] = z / jnp.maximum(nrm, 1e-12)


def _row_blocks(feat):
    return pl.BlockSpec((1, ROW_BLK, feat), lambda b, r: (b, r, 0))


def _whole(shape):
    return pl.BlockSpec(shape, lambda b, r: tuple(0 for _ in shape))


_GRID = (2, N_NODES // ROW_BLK)

_dense1 = pl.pallas_call(
    _dense1_body,
    grid=_GRID,
    in_specs=[
        _row_blocks(DIM),
        _row_blocks(1),
        _row_blocks(DIM),
        _whole((DIM, DIM)),
        _whole((1, DIM)),
        _whole((DIM, DIM)),
    ],
    out_specs=_row_blocks(DIM),
    out_shape=jax.ShapeDtypeStruct((2, N_NODES, DIM), jnp.float32),
)

_dense2 = pl.pallas_call(
    _dense2_body,
    grid=_GRID,
    in_specs=[
        _row_blocks(DIM),
        _row_blocks(1),
        _row_blocks(DIM),
        _whole((DIM, DIM)),
        _whole((1, DIM)),
        _whole((DIM, DIM)),
        _whole((DIM, PROJ)),
        _whole((1, PROJ)),
        _whole((PROJ, PROJ)),
        _whole((1, PROJ)),
    ],
    out_specs=[_row_blocks(DIM), _row_blocks(PROJ)],
    out_shape=[
        jax.ShapeDtypeStruct((2, N_NODES, DIM), jnp.float32),
        jax.ShapeDtypeStruct((2, N_NODES, PROJ), jnp.float32),
    ],
)


def _split_edges(ei_row):
    r = ei_row.reshape(NCH, CHUNK)
    base = r[:NUM_SUBCORES * CH_PER_TILE].reshape(NUM_SUBCORES, CH_PER_TILE,
                                                  CHUNK)
    extra = jnp.concatenate(
        [r[NUM_SUBCORES * CH_PER_TILE:],
         jnp.zeros((NUM_SUBCORES - CH_EXTRA, CHUNK), jnp.int32)], axis=0)
    return jnp.concatenate([base, extra[:, None, :]], axis=1)


def _split_feat(xs):
    return jnp.stack([xs[:, :, :HALF], xs[:, :, HALF:]], axis=1)


def _join_feat(agg):
    return jnp.concatenate([agg[:, 0], agg[:, 1]], axis=-1)


def kernel(A_x, B_x, A_edge_index, B_edge_index,
           Wl1, bl1, Wr1, Wl2, bl2, Wr2, Pw1, Pb1, Pw2, Pb2):
    xs = jnp.stack([A_x, B_x])
    srcs = jnp.stack([_split_edges(A_edge_index[0]),
                      _split_edges(B_edge_index[0])])
    dsts = jnp.stack([_split_edges(A_edge_index[1]),
                      _split_edges(B_edge_index[1])])
    zrows = jnp.zeros((RPT, HALF), jnp.float32)
    zcnt = jnp.zeros((N_NODES,), jnp.float32)

    agg1, cnt = _make_segsum(True)(_split_feat(xs), srcs, dsts, zrows, zcnt)
    cnt3 = cnt[:, :, None]
    h1 = _dense1(_join_feat(agg1), cnt3, xs, Wl1.T, bl1[None], Wr1.T)
    (agg2,) = _make_segsum(False)(_split_feat(h1), srcs, dsts, zrows, zcnt)
    h, z = _dense2(_join_feat(agg2), cnt3, h1, Wl2.T, bl2[None], Wr2.T,
                   Pw1.T, Pb1[None], Pw2.T, Pb2[None])
    return (h[0], h[1], z[0], z[1])


# speedup vs baseline: 5.0968x; 1.2420x over previous
import functools

import jax
import jax.numpy as jnp
from jax import lax
from jax.experimental import pallas as pl
from jax.experimental.pallas import tpu as pltpu
from jax.experimental.pallas import tpu_sc as plsc

N_NODES = 10000
N_EDGES = 320000
DIM = 128
HALF = DIM // 2
PROJ = 64
NUM_CORES = 2
NUM_SUBCORES = 16
CHUNK = 128
NCH = N_EDGES // CHUNK
CH_PER_TILE = NCH // NUM_SUBCORES
CH_EXTRA = NCH - CH_PER_TILE * NUM_SUBCORES
CH_BUF = CH_PER_TILE + 1
RPT = 632
RPT_LAST = N_NODES - (NUM_SUBCORES - 1) * RPT
ROW_BLK = 2000


def _make_segsum(with_cnt: bool):
    mesh = plsc.VectorSubcoreMesh(core_axis_name="c", subcore_axis_name="s",
                                  num_cores=NUM_CORES, num_subcores=NUM_SUBCORES)
    out_type = [jax.ShapeDtypeStruct((NUM_CORES, 2, N_NODES, HALF),
                                     jnp.float32)]
    if with_cnt:
        out_type.append(jax.ShapeDtypeStruct((NUM_CORES, N_NODES), jnp.float32))
    scratch = [
        pltpu.VMEM((CH_BUF, CHUNK), jnp.int32),
        pltpu.VMEM((CH_BUF, CHUNK), jnp.int32),
        pltpu.VMEM((CHUNK, HALF), jnp.float32),
        pltpu.VMEM((CHUNK, HALF), jnp.float32),
        pltpu.VMEM((CHUNK,), jnp.float32),
        pltpu.VMEM_SHARED((N_NODES, HALF), jnp.float32),
        pltpu.VMEM_SHARED((N_NODES,), jnp.float32),
        pltpu.SemaphoreType.DMA,
        pltpu.SemaphoreType.DMA,
        pltpu.SemaphoreType.DMA,
        pltpu.SemaphoreType.DMA,
        pltpu.SemaphoreType.DMA,
        pltpu.SemaphoreType.DMA,
    ]

    def body(x_hbm, src_hbm, dst_hbm, zrows_hbm, zcnt_hbm, agg_hbm, *rest):
        if with_cnt:
            cnt_hbm = rest[0]
            rest = rest[1:]
        (src_v, dst_v, rows0, rows1, ones_v, acc_sh, cnt_sh,
         g0, g1, s0, s1, c0, c1) = rest
        core = lax.axis_index("c")
        sub = lax.axis_index("s")

        def zero_acc():
            @pl.when(sub < NUM_SUBCORES - 1)
            def _zero_full():
                pltpu.sync_copy(zrows_hbm, acc_sh.at[pl.ds(sub * RPT, RPT)])

            @pl.when(sub == NUM_SUBCORES - 1)
            def _zero_last():
                pltpu.sync_copy(
                    zrows_hbm.at[pl.ds(0, RPT_LAST)],
                    acc_sh.at[pl.ds((NUM_SUBCORES - 1) * RPT, RPT_LAST)])

        def write_acc(half):
            @pl.when(sub < NUM_SUBCORES - 1)
            def _out_full():
                pltpu.sync_copy(acc_sh.at[pl.ds(sub * RPT, RPT)],
                                agg_hbm.at[core].at[half]
                                .at[pl.ds(sub * RPT, RPT)])

            @pl.when(sub == NUM_SUBCORES - 1)
            def _out_last():
                pltpu.sync_copy(
                    acc_sh.at[pl.ds((NUM_SUBCORES - 1) * RPT, RPT_LAST)],
                    agg_hbm.at[core].at[half]
                    .at[pl.ds((NUM_SUBCORES - 1) * RPT, RPT_LAST)])

        if with_cnt:
            @pl.when(sub == 0)
            def _zero_cnt():
                pltpu.sync_copy(zcnt_hbm, cnt_sh)
            for i in range(CHUNK // 16):
                ones_v[pl.ds(i * 16, 16)] = jnp.ones((16,), jnp.float32)

        pltpu.sync_copy(src_hbm.at[core].at[sub], src_v)
        pltpu.sync_copy(dst_hbm.at[core].at[sub], dst_v)

        nch = CH_PER_TILE + jnp.where(sub < CH_EXTRA, 1, 0)
        zero_acc()
        plsc.subcore_barrier()

        npairs = CH_PER_TILE // 2

        for half in range(2):
            cnt_here = with_cnt and half == 0
            table = x_hbm.at[core].at[half]

            def g_start(j, buf, sem):
                pltpu.async_copy(table.at[src_v.at[j]], buf, sem)

            def g_wait(buf, sem):
                pltpu.make_async_copy(table.at[src_v.at[0]], buf, sem).wait()

            def s_start(buf, j, sem):
                pltpu.async_copy(buf, acc_sh.at[dst_v.at[j]], sem, add=True)

            def s_wait(buf, sem):
                pltpu.make_async_copy(buf, acc_sh.at[dst_v.at[0]], sem).wait()

            def c_start(j, sem):
                pltpu.async_copy(ones_v, cnt_sh.at[dst_v.at[j]], sem, add=True)

            def c_wait(sem):
                pltpu.make_async_copy(ones_v, cnt_sh.at[dst_v.at[0]],
                                      sem).wait()

            g_start(0, rows0, g0)

            def pair(i, carry):
                j0 = 2 * i
                g_wait(rows0, g0)

                @pl.when(i > 0)
                def _drain_prev():
                    s_wait(rows1, s1)
                    if cnt_here:
                        c_wait(c0)
                        c_wait(c1)

                g_start(j0 + 1, rows1, g1)
                s_start(rows0, j0, s0)
                if cnt_here:
                    c_start(j0, c0)
                g_wait(rows1, g1)
                s_wait(rows0, s0)

                @pl.when(j0 + 2 < nch)
                def _next_gather():
                    g_start(j0 + 2, rows0, g0)

                s_start(rows1, j0 + 1, s1)
                if cnt_here:
                    c_start(j0 + 1, c1)
                return carry

            lax.fori_loop(0, npairs, pair, 0)
            s_wait(rows1, s1)
            if cnt_here:
                c_wait(c0)
                c_wait(c1)

            @pl.when(nch > CH_PER_TILE)
            def _extra_chunk():
                g_wait(rows0, g0)
                pltpu.sync_copy(rows0, acc_sh.at[dst_v.at[CH_PER_TILE]],
                                add=True)
                if cnt_here:
                    pltpu.sync_copy(ones_v, cnt_sh.at[dst_v.at[CH_PER_TILE]],
                                    add=True)

            plsc.subcore_barrier()
            write_acc(half)
            if half == 0:
                if with_cnt:
                    @pl.when(sub == 0)
                    def _out_cnt():
                        pltpu.sync_copy(cnt_sh, cnt_hbm.at[core])
                zero_acc()
                plsc.subcore_barrier()

    return pl.kernel(body, out_type=tuple(out_type), mesh=mesh,
                     scratch_types=scratch,
                     compiler_params=pltpu.CompilerParams(
                         use_tc_tiling_on_sc=False))


_make_segsum = functools.lru_cache(maxsize=None)(_make_segsum)


def _dense1_body(agg_ref, cnt_ref, x_ref, wl_ref, bl_ref, wr_ref, h_ref):
    a = agg_ref[0] / jnp.maximum(cnt_ref[0], 1.0)
    h = (jnp.dot(a, wl_ref[...], preferred_element_type=jnp.float32)
         + bl_ref[...]
         + jnp.dot(x_ref[0], wr_ref[...], preferred_element_type=jnp.float32))
    h_ref[0] = jnp.maximum(h, 0.0)


def _dense2_body(agg_ref, cnt_ref, x_ref, wl_ref, bl_ref, wr_ref,
                 pw1_ref, pb1_ref, pw2_ref, pb2_ref, h_ref, z_ref):
    a = agg_ref[0] / jnp.maximum(cnt_ref[0], 1.0)
    h = (jnp.dot(a, wl_ref[...], preferred_element_type=jnp.float32)
         + bl_ref[...]
         + jnp.dot(x_ref[0], wr_ref[...], preferred_element_type=jnp.float32))
    h_ref[0] = h
    t = jnp.maximum(
        jnp.dot(h, pw1_ref[...], preferred_element_type=jnp.float32)
        + pb1_ref[...], 0.0)
    z = (jnp.dot(t, pw2_ref[...], preferred_element_type=jnp.float32)
         + pb2_ref[...])
    nrm = jnp.sqrt(jnp.sum(z * z, axis=1, keepdims=True))
    z_ref[0] = z / jnp.maximum(nrm, 1e-12)


def _row_blocks(feat):
    return pl.BlockSpec((1, ROW_BLK, feat), lambda b, r: (b, r, 0))


def _whole(shape):
    return pl.BlockSpec(shape, lambda b, r: tuple(0 for _ in shape))


_GRID = (2, N_NODES // ROW_BLK)

_dense1 = pl.pallas_call(
    _dense1_body,
    grid=_GRID,
    in_specs=[
        _row_blocks(DIM),
        _row_blocks(1),
        _row_blocks(DIM),
        _whole((DIM, DIM)),
        _whole((1, DIM)),
        _whole((DIM, DIM)),
    ],
    out_specs=_row_blocks(DIM),
    out_shape=jax.ShapeDtypeStruct((2, N_NODES, DIM), jnp.float32),
)

_dense2 = pl.pallas_call(
    _dense2_body,
    grid=_GRID,
    in_specs=[
        _row_blocks(DIM),
        _row_blocks(1),
        _row_blocks(DIM),
        _whole((DIM, DIM)),
        _whole((1, DIM)),
        _whole((DIM, DIM)),
        _whole((DIM, PROJ)),
        _whole((1, PROJ)),
        _whole((PROJ, PROJ)),
        _whole((1, PROJ)),
    ],
    out_specs=[_row_blocks(DIM), _row_blocks(PROJ)],
    out_shape=[
        jax.ShapeDtypeStruct((2, N_NODES, DIM), jnp.float32),
        jax.ShapeDtypeStruct((2, N_NODES, PROJ), jnp.float32),
    ],
)


def _split_edges(ei_row):
    r = ei_row.reshape(NCH, CHUNK)
    base = r[:NUM_SUBCORES * CH_PER_TILE].reshape(NUM_SUBCORES, CH_PER_TILE,
                                                  CHUNK)
    extra = jnp.concatenate(
        [r[NUM_SUBCORES * CH_PER_TILE:],
         jnp.zeros((NUM_SUBCORES - CH_EXTRA, CHUNK), jnp.int32)], axis=0)
    return jnp.concatenate([base, extra[:, None, :]], axis=1)


def _split_feat(xs):
    return jnp.stack([xs[:, :, :HALF], xs[:, :, HALF:]], axis=1)


def _join_feat(agg):
    return jnp.concatenate([agg[:, 0], agg[:, 1]], axis=-1)


def kernel(A_x, B_x, A_edge_index, B_edge_index,
           Wl1, bl1, Wr1, Wl2, bl2, Wr2, Pw1, Pb1, Pw2, Pb2):
    xs = jnp.stack([A_x, B_x])
    srcs = jnp.stack([_split_edges(A_edge_index[0]),
                      _split_edges(B_edge_index[0])])
    dsts = jnp.stack([_split_edges(A_edge_index[1]),
                      _split_edges(B_edge_index[1])])
    zrows = jnp.zeros((RPT, HALF), jnp.float32)
    zcnt = jnp.zeros((N_NODES,), jnp.float32)

    agg1, cnt = _make_segsum(True)(_split_feat(xs), srcs, dsts, zrows, zcnt)
    cnt3 = cnt[:, :, None]
    h1 = _dense1(_join_feat(agg1), cnt3, xs, Wl1.T, bl1[None], Wr1.T)
    (agg2,) = _make_segsum(False)(_split_feat(h1), srcs, dsts, zrows, zcnt)
    h, z = _dense2(_join_feat(agg2), cnt3, h1, Wl2.T, bl2[None], Wr2.T,
                   Pw1.T, Pb1[None], Pw2.T, Pb2[None])
    return (h[0], h[1], z[0], z[1])


# speedup vs baseline: 7.1093x; 1.3948x over previous
import functools

import jax
import jax.numpy as jnp
from jax import lax
from jax.experimental import pallas as pl
from jax.experimental.pallas import tpu as pltpu
from jax.experimental.pallas import tpu_sc as plsc

N_NODES = 10000
N_EDGES = 320000
DIM = 128
HALF = DIM // 2
PROJ = 64
NUM_CORES = 2
NUM_SUBCORES = 16
CHUNK = 128
NCH = N_EDGES // CHUNK
CH_PER_TILE = NCH // NUM_SUBCORES
CH_EXTRA = NCH - CH_PER_TILE * NUM_SUBCORES
CH_BUF = CH_PER_TILE + 1
RPT = 632
RPT_LAST = N_NODES - (NUM_SUBCORES - 1) * RPT
NBUF = 4
AHEAD = 2
ROW_BLK = 2000


def _make_segsum(with_cnt: bool):
    mesh = plsc.VectorSubcoreMesh(core_axis_name="c", subcore_axis_name="s",
                                  num_cores=NUM_CORES, num_subcores=NUM_SUBCORES)
    out_type = [jax.ShapeDtypeStruct((NUM_CORES, 2, N_NODES, HALF),
                                     jnp.float32)]
    if with_cnt:
        out_type.append(jax.ShapeDtypeStruct((NUM_CORES, N_NODES), jnp.float32))
    scratch = (
        [pltpu.VMEM((CH_BUF, CHUNK), jnp.int32),
         pltpu.VMEM((CH_BUF, CHUNK), jnp.int32)]
        + [pltpu.VMEM((CHUNK, HALF), jnp.float32)] * NBUF
        + [pltpu.VMEM((CHUNK,), jnp.float32),
           pltpu.VMEM_SHARED((N_NODES, HALF), jnp.float32),
           pltpu.VMEM_SHARED((N_NODES,), jnp.float32)]
        + [pltpu.SemaphoreType.DMA] * (3 * NBUF)
    )

    def body(x_hbm, src_hbm, dst_hbm, zrows_hbm, zcnt_hbm, agg_hbm, *rest):
        if with_cnt:
            cnt_hbm = rest[0]
            rest = rest[1:]
        src_v, dst_v = rest[0], rest[1]
        rows = rest[2:2 + NBUF]
        ones_v, acc_sh, cnt_sh = rest[2 + NBUF:5 + NBUF]
        sems = rest[5 + NBUF:]
        gsem, ssem, csem = (sems[:NBUF], sems[NBUF:2 * NBUF],
                            sems[2 * NBUF:3 * NBUF])
        core = lax.axis_index("c")
        sub = lax.axis_index("s")

        def zero_acc():
            @pl.when(sub < NUM_SUBCORES - 1)
            def _zero_full():
                pltpu.sync_copy(zrows_hbm, acc_sh.at[pl.ds(sub * RPT, RPT)])

            @pl.when(sub == NUM_SUBCORES - 1)
            def _zero_last():
                pltpu.sync_copy(
                    zrows_hbm.at[pl.ds(0, RPT_LAST)],
                    acc_sh.at[pl.ds((NUM_SUBCORES - 1) * RPT, RPT_LAST)])

        def write_acc(half):
            @pl.when(sub < NUM_SUBCORES - 1)
            def _out_full():
                pltpu.sync_copy(acc_sh.at[pl.ds(sub * RPT, RPT)],
                                agg_hbm.at[core].at[half]
                                .at[pl.ds(sub * RPT, RPT)])

            @pl.when(sub == NUM_SUBCORES - 1)
            def _out_last():
                pltpu.sync_copy(
                    acc_sh.at[pl.ds((NUM_SUBCORES - 1) * RPT, RPT_LAST)],
                    agg_hbm.at[core].at[half]
                    .at[pl.ds((NUM_SUBCORES - 1) * RPT, RPT_LAST)])

        if with_cnt:
            @pl.when(sub == 0)
            def _zero_cnt():
                pltpu.sync_copy(zcnt_hbm, cnt_sh)
            for i in range(CHUNK // 16):
                ones_v[pl.ds(i * 16, 16)] = jnp.ones((16,), jnp.float32)

        pltpu.sync_copy(src_hbm.at[core].at[sub], src_v)
        pltpu.sync_copy(dst_hbm.at[core].at[sub], dst_v)

        nch = CH_PER_TILE + jnp.where(sub < CH_EXTRA, 1, 0)
        zero_acc()
        plsc.subcore_barrier()

        nquad = CH_PER_TILE // NBUF

        for half in range(2):
            cnt_here = with_cnt and half == 0
            table = x_hbm.at[core].at[half]

            def g_start(j, k):
                pltpu.async_copy(table.at[src_v.at[j]], rows[k], gsem[k])

            def g_wait(k):
                pltpu.make_async_copy(table.at[src_v.at[0]], rows[k],
                                      gsem[k]).wait()

            def s_start(j, k):
                pltpu.async_copy(rows[k], acc_sh.at[dst_v.at[j]], ssem[k],
                                 add=True)

            def s_wait(k):
                pltpu.make_async_copy(rows[k], acc_sh.at[dst_v.at[0]],
                                      ssem[k]).wait()

            def c_start(j, k):
                pltpu.async_copy(ones_v, cnt_sh.at[dst_v.at[j]], csem[k],
                                 add=True)

            def c_wait(k):
                pltpu.make_async_copy(ones_v, cnt_sh.at[dst_v.at[0]],
                                      csem[k]).wait()

            g_start(0, 0)
            g_start(1, 1)

            def ring(i, carry):
                for k in range(NBUF):
                    j = NBUF * i + k
                    kd = (k + AHEAD) % NBUF

                    @pl.when(j >= AHEAD)
                    def _drain():
                        s_wait(kd)
                        if cnt_here:
                            c_wait(kd)

                    @pl.when(j + AHEAD < nch)
                    def _prefetch():
                        g_start(j + AHEAD, kd)

                    g_wait(k)
                    s_start(j, k)
                    if cnt_here:
                        c_start(j, k)
                return carry

            lax.fori_loop(0, nquad, ring, 0)
            for k in range(NBUF - AHEAD, NBUF):
                s_wait(k)
                if cnt_here:
                    c_wait(k)

            @pl.when(nch > CH_PER_TILE)
            def _extra_chunk():
                g_wait(CH_PER_TILE % NBUF)
                pltpu.sync_copy(rows[CH_PER_TILE % NBUF],
                                acc_sh.at[dst_v.at[CH_PER_TILE]], add=True)
                if cnt_here:
                    pltpu.sync_copy(ones_v, cnt_sh.at[dst_v.at[CH_PER_TILE]],
                                    add=True)

            plsc.subcore_barrier()
            write_acc(half)
            if half == 0:
                if with_cnt:
                    @pl.when(sub == 0)
                    def _out_cnt():
                        pltpu.sync_copy(cnt_sh, cnt_hbm.at[core])
                zero_acc()
                plsc.subcore_barrier()

    return pl.kernel(body, out_type=tuple(out_type), mesh=mesh,
                     scratch_types=scratch,
                     compiler_params=pltpu.CompilerParams(
                         use_tc_tiling_on_sc=False))


_make_segsum = functools.lru_cache(maxsize=None)(_make_segsum)


def _dense1_body(agg_ref, cnt_ref, x_ref, wl_ref, bl_ref, wr_ref, h_ref):
    a = agg_ref[0] / jnp.maximum(cnt_ref[0], 1.0)
    h = (jnp.dot(a, wl_ref[...], preferred_element_type=jnp.float32)
         + bl_ref[...]
         + jnp.dot(x_ref[0], wr_ref[...], preferred_element_type=jnp.float32))
    h_ref[0] = jnp.maximum(h, 0.0)


def _dense2_body(agg_ref, cnt_ref, x_ref, wl_ref, bl_ref, wr_ref,
                 pw1_ref, pb1_ref, pw2_ref, pb2_ref, h_ref, z_ref):
    a = agg_ref[0] / jnp.maximum(cnt_ref[0], 1.0)
    h = (jnp.dot(a, wl_ref[...], preferred_element_type=jnp.float32)
         + bl_ref[...]
         + jnp.dot(x_ref[0], wr_ref[...], preferred_element_type=jnp.float32))
    h_ref[0] = h
    t = jnp.maximum(
        jnp.dot(h, pw1_ref[...], preferred_element_type=jnp.float32)
        + pb1_ref[...], 0.0)
    z = (jnp.dot(t, pw2_ref[...], preferred_element_type=jnp.float32)
         + pb2_ref[...])
    nrm = jnp.sqrt(jnp.sum(z * z, axis=1, keepdims=True))
    z_ref[0] = z / jnp.maximum(nrm, 1e-12)


def _row_blocks(feat):
    return pl.BlockSpec((1, ROW_BLK, feat), lambda b, r: (b, r, 0))


def _whole(shape):
    return pl.BlockSpec(shape, lambda b, r: tuple(0 for _ in shape))


_GRID = (2, N_NODES // ROW_BLK)

_dense1 = pl.pallas_call(
    _dense1_body,
    grid=_GRID,
    in_specs=[
        _row_blocks(DIM),
        _row_blocks(1),
        _row_blocks(DIM),
        _whole((DIM, DIM)),
        _whole((1, DIM)),
        _whole((DIM, DIM)),
    ],
    out_specs=_row_blocks(DIM),
    out_shape=jax.ShapeDtypeStruct((2, N_NODES, DIM), jnp.float32),
)

_dense2 = pl.pallas_call(
    _dense2_body,
    grid=_GRID,
    in_specs=[
        _row_blocks(DIM),
        _row_blocks(1),
        _row_blocks(DIM),
        _whole((DIM, DIM)),
        _whole((1, DIM)),
        _whole((DIM, DIM)),
        _whole((DIM, PROJ)),
        _whole((1, PROJ)),
        _whole((PROJ, PROJ)),
        _whole((1, PROJ)),
    ],
    out_specs=[_row_blocks(DIM), _row_blocks(PROJ)],
    out_shape=[
        jax.ShapeDtypeStruct((2, N_NODES, DIM), jnp.float32),
        jax.ShapeDtypeStruct((2, N_NODES, PROJ), jnp.float32),
    ],
)


def _split_edges(ei_row):
    r = ei_row.reshape(NCH, CHUNK)
    base = r[:NUM_SUBCORES * CH_PER_TILE].reshape(NUM_SUBCORES, CH_PER_TILE,
                                                  CHUNK)
    extra = jnp.concatenate(
        [r[NUM_SUBCORES * CH_PER_TILE:],
         jnp.zeros((NUM_SUBCORES - CH_EXTRA, CHUNK), jnp.int32)], axis=0)
    return jnp.concatenate([base, extra[:, None, :]], axis=1)


def _split_feat(xs):
    return jnp.stack([xs[:, :, :HALF], xs[:, :, HALF:]], axis=1)


def _join_feat(agg):
    return jnp.concatenate([agg[:, 0], agg[:, 1]], axis=-1)


def kernel(A_x, B_x, A_edge_index, B_edge_index,
           Wl1, bl1, Wr1, Wl2, bl2, Wr2, Pw1, Pb1, Pw2, Pb2):
    xs = jnp.stack([A_x, B_x])
    srcs = jnp.stack([_split_edges(A_edge_index[0]),
                      _split_edges(B_edge_index[0])])
    dsts = jnp.stack([_split_edges(A_edge_index[1]),
                      _split_edges(B_edge_index[1])])
    zrows = jnp.zeros((RPT, HALF), jnp.float32)
    zcnt = jnp.zeros((N_NODES,), jnp.float32)

    agg1, cnt = _make_segsum(True)(_split_feat(xs), srcs, dsts, zrows, zcnt)
    cnt3 = cnt[:, :, None]
    h1 = _dense1(_join_feat(agg1), cnt3, xs, Wl1.T, bl1[None], Wr1.T)
    (agg2,) = _make_segsum(False)(_split_feat(h1), srcs, dsts, zrows, zcnt)
    h, z = _dense2(_join_feat(agg2), cnt3, h1, Wl2.T, bl2[None], Wr2.T,
                   Pw1.T, Pb1[None], Pw2.T, Pb2[None])
    return (h[0], h[1], z[0], z[1])


# speedup vs baseline: 7.4348x; 1.0458x over previous
import functools

import jax
import jax.numpy as jnp
from jax import lax
from jax.experimental import pallas as pl
from jax.experimental.pallas import tpu as pltpu
from jax.experimental.pallas import tpu_sc as plsc

N_NODES = 10000
N_EDGES = 320000
DIM = 128
HALF = DIM // 2
PROJ = 64
NUM_CORES = 2
NUM_SUBCORES = 16
CHUNK = 128
NCH = N_EDGES // CHUNK
CH_PER_TILE = NCH // NUM_SUBCORES
CH_EXTRA = NCH - CH_PER_TILE * NUM_SUBCORES
CH_BUF = CH_PER_TILE + 1
RPT = 632
RPT_LAST = N_NODES - (NUM_SUBCORES - 1) * RPT
NBUF = 6
AHEAD = 3
ROW_BLK = 2000


def _make_segsum(with_cnt: bool):
    mesh = plsc.VectorSubcoreMesh(core_axis_name="c", subcore_axis_name="s",
                                  num_cores=NUM_CORES, num_subcores=NUM_SUBCORES)
    out_type = [jax.ShapeDtypeStruct((NUM_CORES, 2, N_NODES, HALF),
                                     jnp.float32)]
    if with_cnt:
        out_type.append(jax.ShapeDtypeStruct((NUM_CORES, N_NODES), jnp.float32))
    scratch = (
        [pltpu.VMEM((CH_BUF, CHUNK), jnp.int32),
         pltpu.VMEM((CH_BUF, CHUNK), jnp.int32)]
        + [pltpu.VMEM((CHUNK, HALF), jnp.float32)] * NBUF
        + [pltpu.VMEM((CHUNK,), jnp.float32),
           pltpu.VMEM_SHARED((N_NODES, HALF), jnp.float32),
           pltpu.VMEM_SHARED((N_NODES,), jnp.float32)]
        + [pltpu.SemaphoreType.DMA] * (3 * NBUF)
    )

    def body(x_hbm, src_hbm, dst_hbm, zrows_hbm, zcnt_hbm, agg_hbm, *rest):
        if with_cnt:
            cnt_hbm = rest[0]
            rest = rest[1:]
        src_v, dst_v = rest[0], rest[1]
        rows = rest[2:2 + NBUF]
        ones_v, acc_sh, cnt_sh = rest[2 + NBUF:5 + NBUF]
        sems = rest[5 + NBUF:]
        gsem, ssem, csem = (sems[:NBUF], sems[NBUF:2 * NBUF],
                            sems[2 * NBUF:3 * NBUF])
        core = lax.axis_index("c")
        sub = lax.axis_index("s")

        def zero_acc():
            @pl.when(sub < NUM_SUBCORES - 1)
            def _zero_full():
                pltpu.sync_copy(zrows_hbm, acc_sh.at[pl.ds(sub * RPT, RPT)])

            @pl.when(sub == NUM_SUBCORES - 1)
            def _zero_last():
                pltpu.sync_copy(
                    zrows_hbm.at[pl.ds(0, RPT_LAST)],
                    acc_sh.at[pl.ds((NUM_SUBCORES - 1) * RPT, RPT_LAST)])

        def write_acc(half):
            @pl.when(sub < NUM_SUBCORES - 1)
            def _out_full():
                pltpu.sync_copy(acc_sh.at[pl.ds(sub * RPT, RPT)],
                                agg_hbm.at[core].at[half]
                                .at[pl.ds(sub * RPT, RPT)])

            @pl.when(sub == NUM_SUBCORES - 1)
            def _out_last():
                pltpu.sync_copy(
                    acc_sh.at[pl.ds((NUM_SUBCORES - 1) * RPT, RPT_LAST)],
                    agg_hbm.at[core].at[half]
                    .at[pl.ds((NUM_SUBCORES - 1) * RPT, RPT_LAST)])

        if with_cnt:
            @pl.when(sub == 0)
            def _zero_cnt():
                pltpu.sync_copy(zcnt_hbm, cnt_sh)
            for i in range(CHUNK // 16):
                ones_v[pl.ds(i * 16, 16)] = jnp.ones((16,), jnp.float32)

        pltpu.sync_copy(src_hbm.at[core].at[sub], src_v)
        pltpu.sync_copy(dst_hbm.at[core].at[sub], dst_v)

        nch = CH_PER_TILE + jnp.where(sub < CH_EXTRA, 1, 0)
        zero_acc()
        plsc.subcore_barrier()

        nquad = CH_PER_TILE // NBUF

        for half in range(2):
            cnt_here = with_cnt and half == 0
            table = x_hbm.at[core].at[half]

            def g_start(j, k):
                pltpu.async_copy(table.at[src_v.at[j]], rows[k], gsem[k])

            def g_wait(k):
                pltpu.make_async_copy(table.at[src_v.at[0]], rows[k],
                                      gsem[k]).wait()

            def s_start(j, k):
                pltpu.async_copy(rows[k], acc_sh.at[dst_v.at[j]], ssem[k],
                                 add=True)

            def s_wait(k):
                pltpu.make_async_copy(rows[k], acc_sh.at[dst_v.at[0]],
                                      ssem[k]).wait()

            def c_start(j, k):
                pltpu.async_copy(ones_v, cnt_sh.at[dst_v.at[j]], csem[k],
                                 add=True)

            def c_wait(k):
                pltpu.make_async_copy(ones_v, cnt_sh.at[dst_v.at[0]],
                                      csem[k]).wait()

            for j0 in range(AHEAD):
                g_start(j0, j0)

            def ring(i, carry):
                for k in range(NBUF):
                    j = NBUF * i + k
                    kd = (k + AHEAD) % NBUF

                    @pl.when(j >= AHEAD)
                    def _drain():
                        s_wait(kd)
                        if cnt_here:
                            c_wait(kd)

                    @pl.when(j + AHEAD < nch)
                    def _prefetch():
                        g_start(j + AHEAD, kd)

                    g_wait(k)
                    s_start(j, k)
                    if cnt_here:
                        c_start(j, k)
                return carry

            lax.fori_loop(0, nquad, ring, 0)
            for k in range(NBUF - AHEAD, NBUF):
                s_wait(k)
                if cnt_here:
                    c_wait(k)

            @pl.when(nch > CH_PER_TILE)
            def _extra_chunk():
                g_wait(CH_PER_TILE % NBUF)
                pltpu.sync_copy(rows[CH_PER_TILE % NBUF],
                                acc_sh.at[dst_v.at[CH_PER_TILE]], add=True)
                if cnt_here:
                    pltpu.sync_copy(ones_v, cnt_sh.at[dst_v.at[CH_PER_TILE]],
                                    add=True)

            plsc.subcore_barrier()
            write_acc(half)
            if half == 0:
                if with_cnt:
                    @pl.when(sub == 0)
                    def _out_cnt():
                        pltpu.sync_copy(cnt_sh, cnt_hbm.at[core])
                zero_acc()
                plsc.subcore_barrier()

    return pl.kernel(body, out_type=tuple(out_type), mesh=mesh,
                     scratch_types=scratch,
                     compiler_params=pltpu.CompilerParams(
                         use_tc_tiling_on_sc=False))


_make_segsum = functools.lru_cache(maxsize=None)(_make_segsum)


def _dense1_body(agg_ref, cnt_ref, x_ref, wl_ref, bl_ref, wr_ref, h_ref):
    a = agg_ref[0] / jnp.maximum(cnt_ref[0], 1.0)
    h = (jnp.dot(a, wl_ref[...], preferred_element_type=jnp.float32)
         + bl_ref[...]
         + jnp.dot(x_ref[0], wr_ref[...], preferred_element_type=jnp.float32))
    h_ref[0] = jnp.maximum(h, 0.0)


def _dense2_body(agg_ref, cnt_ref, x_ref, wl_ref, bl_ref, wr_ref,
                 pw1_ref, pb1_ref, pw2_ref, pb2_ref, h_ref, z_ref):
    a = agg_ref[0] / jnp.maximum(cnt_ref[0], 1.0)
    h = (jnp.dot(a, wl_ref[...], preferred_element_type=jnp.float32)
         + bl_ref[...]
         + jnp.dot(x_ref[0], wr_ref[...], preferred_element_type=jnp.float32))
    h_ref[0] = h
    t = jnp.maximum(
        jnp.dot(h, pw1_ref[...], preferred_element_type=jnp.float32)
        + pb1_ref[...], 0.0)
    z = (jnp.dot(t, pw2_ref[...], preferred_element_type=jnp.float32)
         + pb2_ref[...])
    nrm = jnp.sqrt(jnp.sum(z * z, axis=1, keepdims=True))
    z_ref[0] = z / jnp.maximum(nrm, 1e-12)


def _row_blocks(feat):
    return pl.BlockSpec((1, ROW_BLK, feat), lambda b, r: (b, r, 0))


def _whole(shape):
    return pl.BlockSpec(shape, lambda b, r: tuple(0 for _ in shape))


_GRID = (2, N_NODES // ROW_BLK)

_dense1 = pl.pallas_call(
    _dense1_body,
    grid=_GRID,
    in_specs=[
        _row_blocks(DIM),
        _row_blocks(1),
        _row_blocks(DIM),
        _whole((DIM, DIM)),
        _whole((1, DIM)),
        _whole((DIM, DIM)),
    ],
    out_specs=_row_blocks(DIM),
    out_shape=jax.ShapeDtypeStruct((2, N_NODES, DIM), jnp.float32),
)

_dense2 = pl.pallas_call(
    _dense2_body,
    grid=_GRID,
    in_specs=[
        _row_blocks(DIM),
        _row_blocks(1),
        _row_blocks(DIM),
        _whole((DIM, DIM)),
        _whole((1, DIM)),
        _whole((DIM, DIM)),
        _whole((DIM, PROJ)),
        _whole((1, PROJ)),
        _whole((PROJ, PROJ)),
        _whole((1, PROJ)),
    ],
    out_specs=[_row_blocks(DIM), _row_blocks(PROJ)],
    out_shape=[
        jax.ShapeDtypeStruct((2, N_NODES, DIM), jnp.float32),
        jax.ShapeDtypeStruct((2, N_NODES, PROJ), jnp.float32),
    ],
)


def _split_edges(ei_row):
    r = ei_row.reshape(NCH, CHUNK)
    base = r[:NUM_SUBCORES * CH_PER_TILE].reshape(NUM_SUBCORES, CH_PER_TILE,
                                                  CHUNK)
    extra = jnp.concatenate(
        [r[NUM_SUBCORES * CH_PER_TILE:],
         jnp.zeros((NUM_SUBCORES - CH_EXTRA, CHUNK), jnp.int32)], axis=0)
    return jnp.concatenate([base, extra[:, None, :]], axis=1)


def _split_feat(xs):
    return jnp.stack([xs[:, :, :HALF], xs[:, :, HALF:]], axis=1)


def _join_feat(agg):
    return jnp.concatenate([agg[:, 0], agg[:, 1]], axis=-1)


def kernel(A_x, B_x, A_edge_index, B_edge_index,
           Wl1, bl1, Wr1, Wl2, bl2, Wr2, Pw1, Pb1, Pw2, Pb2):
    xs = jnp.stack([A_x, B_x])
    srcs = jnp.stack([_split_edges(A_edge_index[0]),
                      _split_edges(B_edge_index[0])])
    dsts = jnp.stack([_split_edges(A_edge_index[1]),
                      _split_edges(B_edge_index[1])])
    zrows = jnp.zeros((RPT, HALF), jnp.float32)
    zcnt = jnp.zeros((N_NODES,), jnp.float32)

    agg1, cnt = _make_segsum(True)(_split_feat(xs), srcs, dsts, zrows, zcnt)
    cnt3 = cnt[:, :, None]
    h1 = _dense1(_join_feat(agg1), cnt3, xs, Wl1.T, bl1[None], Wr1.T)
    (agg2,) = _make_segsum(False)(_split_feat(h1), srcs, dsts, zrows, zcnt)
    h, z = _dense2(_join_feat(agg2), cnt3, h1, Wl2.T, bl2[None], Wr2.T,
                   Pw1.T, Pb1[None], Pw2.T, Pb2[None])
    return (h[0], h[1], z[0], z[1])


# speedup vs baseline: 7.9733x; 1.0724x over previous
import functools

import jax
import jax.numpy as jnp
from jax import lax
from jax.experimental import pallas as pl
from jax.experimental.pallas import tpu as pltpu
from jax.experimental.pallas import tpu_sc as plsc

N_NODES = 10000
N_EDGES = 320000
DIM = 128
HALF = DIM // 2
PROJ = 64
NUM_CORES = 2
NUM_SUBCORES = 16
CHUNK = 128
NCH = N_EDGES // CHUNK
CH_PER_TILE = NCH // NUM_SUBCORES
CH_EXTRA = NCH - CH_PER_TILE * NUM_SUBCORES
CH_BUF = CH_PER_TILE + 1
RPT = 632
RPT_LAST = N_NODES - (NUM_SUBCORES - 1) * RPT
NBUF = 6
AHEAD = 3
ROW_BLK = 2000


def _make_segsum(with_cnt: bool):
    mesh = plsc.VectorSubcoreMesh(core_axis_name="c", subcore_axis_name="s",
                                  num_cores=NUM_CORES, num_subcores=NUM_SUBCORES)
    out_type = [jax.ShapeDtypeStruct((NUM_CORES, 2, N_NODES, HALF),
                                     jnp.float32)]
    if with_cnt:
        out_type.append(jax.ShapeDtypeStruct((NUM_CORES, N_NODES), jnp.float32))
    scratch = (
        [pltpu.VMEM((CH_BUF, CHUNK), jnp.int32),
         pltpu.VMEM((CH_BUF, CHUNK), jnp.int32)]
        + [pltpu.VMEM((CHUNK, HALF), jnp.float32)] * NBUF
        + [pltpu.VMEM((CHUNK,), jnp.float32),
           pltpu.VMEM_SHARED((N_NODES, HALF), jnp.float32),
           pltpu.VMEM_SHARED((N_NODES,), jnp.float32)]
        + [pltpu.SemaphoreType.DMA] * (3 * NBUF)
    )

    def body(x_hbm, src_hbm, dst_hbm, zrows_hbm, zcnt_hbm, agg_hbm, *rest):
        if with_cnt:
            cnt_hbm = rest[0]
            rest = rest[1:]
        src_v, dst_v = rest[0], rest[1]
        rows = rest[2:2 + NBUF]
        ones_v, acc_sh, cnt_sh = rest[2 + NBUF:5 + NBUF]
        sems = rest[5 + NBUF:]
        gsem, ssem, csem = (sems[:NBUF], sems[NBUF:2 * NBUF],
                            sems[2 * NBUF:3 * NBUF])
        core = lax.axis_index("c")
        sub = lax.axis_index("s")

        def zero_acc():
            @pl.when(sub < NUM_SUBCORES - 1)
            def _zero_full():
                pltpu.sync_copy(zrows_hbm, acc_sh.at[pl.ds(sub * RPT, RPT)])

            @pl.when(sub == NUM_SUBCORES - 1)
            def _zero_last():
                pltpu.sync_copy(
                    zrows_hbm.at[pl.ds(0, RPT_LAST)],
                    acc_sh.at[pl.ds((NUM_SUBCORES - 1) * RPT, RPT_LAST)])

        def write_acc(half):
            @pl.when(sub < NUM_SUBCORES - 1)
            def _out_full():
                pltpu.sync_copy(acc_sh.at[pl.ds(sub * RPT, RPT)],
                                agg_hbm.at[core].at[half]
                                .at[pl.ds(sub * RPT, RPT)])

            @pl.when(sub == NUM_SUBCORES - 1)
            def _out_last():
                pltpu.sync_copy(
                    acc_sh.at[pl.ds((NUM_SUBCORES - 1) * RPT, RPT_LAST)],
                    agg_hbm.at[core].at[half]
                    .at[pl.ds((NUM_SUBCORES - 1) * RPT, RPT_LAST)])

        if with_cnt:
            @pl.when(sub == 0)
            def _zero_cnt():
                pltpu.sync_copy(zcnt_hbm, cnt_sh)
            for i in range(CHUNK // 16):
                ones_v[pl.ds(i * 16, 16)] = jnp.ones((16,), jnp.float32)

        pltpu.sync_copy(src_hbm.at[core].at[sub], src_v)
        pltpu.sync_copy(dst_hbm.at[core].at[sub], dst_v)

        nch = CH_PER_TILE + jnp.where(sub < CH_EXTRA, 1, 0)
        zero_acc()
        plsc.subcore_barrier()

        nquad = CH_PER_TILE // NBUF

        for half in range(2):
            cnt_here = with_cnt and half == 0
            table = x_hbm.at[core].at[half]

            def g_start(j, k):
                pltpu.async_copy(table.at[src_v.at[j]], rows[k], gsem[k])

            def g_wait(k):
                pltpu.make_async_copy(table.at[src_v.at[0]], rows[k],
                                      gsem[k]).wait()

            def s_start(j, k):
                pltpu.async_copy(rows[k], acc_sh.at[dst_v.at[j]], ssem[k],
                                 add=True)

            def s_wait(k):
                pltpu.make_async_copy(rows[k], acc_sh.at[dst_v.at[0]],
                                      ssem[k]).wait()

            def c_start(j, k):
                pltpu.async_copy(ones_v, cnt_sh.at[dst_v.at[j]], csem[k],
                                 add=True)

            def c_wait(k):
                pltpu.make_async_copy(ones_v, cnt_sh.at[dst_v.at[0]],
                                      csem[k]).wait()

            for j0 in range(AHEAD):
                g_start(j0, j0)

            def ring(i, carry):
                for k in range(NBUF):
                    j = NBUF * i + k
                    kd = (k + AHEAD) % NBUF

                    @pl.when(j >= AHEAD)
                    def _drain():
                        s_wait(kd)
                        if cnt_here:
                            c_wait(kd)

                    @pl.when(j + AHEAD < nch)
                    def _prefetch():
                        g_start(j + AHEAD, kd)

                    g_wait(k)
                    s_start(j, k)
                    if cnt_here:
                        c_start(j, k)
                return carry

            lax.fori_loop(0, nquad, ring, 0)
            for k in range(NBUF - AHEAD, NBUF):
                s_wait(k)
                if cnt_here:
                    c_wait(k)

            @pl.when(nch > CH_PER_TILE)
            def _extra_chunk():
                g_wait(CH_PER_TILE % NBUF)
                pltpu.sync_copy(rows[CH_PER_TILE % NBUF],
                                acc_sh.at[dst_v.at[CH_PER_TILE]], add=True)
                if cnt_here:
                    pltpu.sync_copy(ones_v, cnt_sh.at[dst_v.at[CH_PER_TILE]],
                                    add=True)

            plsc.subcore_barrier()
            write_acc(half)
            if half == 0:
                if with_cnt:
                    @pl.when(sub == 0)
                    def _out_cnt():
                        pltpu.sync_copy(cnt_sh, cnt_hbm.at[core])
                zero_acc()
                plsc.subcore_barrier()

    return pl.kernel(body, out_type=tuple(out_type), mesh=mesh,
                     scratch_types=scratch,
                     compiler_params=pltpu.CompilerParams(
                         use_tc_tiling_on_sc=False))


_make_segsum = functools.lru_cache(maxsize=None)(_make_segsum)


def _sage(agg_ref, cnt_ref, x_ref, wl_ref, bl_ref, wr_ref):
    inv = 1.0 / jnp.maximum(cnt_ref[0], 1.0)
    return (jnp.dot(agg_ref[0, 0] * inv, wl_ref[:HALF],
                    preferred_element_type=jnp.float32)
            + jnp.dot(agg_ref[0, 1] * inv, wl_ref[HALF:],
                      preferred_element_type=jnp.float32)
            + bl_ref[...]
            + jnp.dot(x_ref[0, 0], wr_ref[:HALF],
                      preferred_element_type=jnp.float32)
            + jnp.dot(x_ref[0, 1], wr_ref[HALF:],
                      preferred_element_type=jnp.float32))


def _dense1_body(agg_ref, cnt_ref, x_ref, wl_ref, bl_ref, wr_ref, h_ref):
    h = jnp.maximum(_sage(agg_ref, cnt_ref, x_ref, wl_ref, bl_ref, wr_ref),
                    0.0)
    h_ref[0, 0] = h[:, :HALF]
    h_ref[0, 1] = h[:, HALF:]


def _dense2_body(agg_ref, cnt_ref, x_ref, wl_ref, bl_ref, wr_ref,
                 pw1_ref, pb1_ref, pw2_ref, pb2_ref, h_ref, z_ref):
    h = _sage(agg_ref, cnt_ref, x_ref, wl_ref, bl_ref, wr_ref)
    h_ref[0] = h
    t = jnp.maximum(
        jnp.dot(h, pw1_ref[...], preferred_element_type=jnp.float32)
        + pb1_ref[...], 0.0)
    z = (jnp.dot(t, pw2_ref[...], preferred_element_type=jnp.float32)
         + pb2_ref[...])
    nrm = jnp.sqrt(jnp.sum(z * z, axis=1, keepdims=True))
    z_ref[0] = z / jnp.maximum(nrm, 1e-12)


def _row_blocks(feat):
    return pl.BlockSpec((1, ROW_BLK, feat), lambda b, r: (b, r, 0))


def _split_blocks():
    return pl.BlockSpec((1, 2, ROW_BLK, HALF), lambda b, r: (b, 0, r, 0))


def _whole(shape):
    return pl.BlockSpec(shape, lambda b, r: tuple(0 for _ in shape))


_GRID = (2, N_NODES // ROW_BLK)

_SPLIT_SHAPE = jax.ShapeDtypeStruct((2, 2, N_NODES, HALF), jnp.float32)

_dense1 = pl.pallas_call(
    _dense1_body,
    grid=_GRID,
    in_specs=[
        _split_blocks(),
        _row_blocks(1),
        _split_blocks(),
        _whole((DIM, DIM)),
        _whole((1, DIM)),
        _whole((DIM, DIM)),
    ],
    out_specs=_split_blocks(),
    out_shape=_SPLIT_SHAPE,
)

_dense2 = pl.pallas_call(
    _dense2_body,
    grid=_GRID,
    in_specs=[
        _split_blocks(),
        _row_blocks(1),
        _split_blocks(),
        _whole((DIM, DIM)),
        _whole((1, DIM)),
        _whole((DIM, DIM)),
        _whole((DIM, PROJ)),
        _whole((1, PROJ)),
        _whole((PROJ, PROJ)),
        _whole((1, PROJ)),
    ],
    out_specs=[_row_blocks(DIM), _row_blocks(PROJ)],
    out_shape=[
        jax.ShapeDtypeStruct((2, N_NODES, DIM), jnp.float32),
        jax.ShapeDtypeStruct((2, N_NODES, PROJ), jnp.float32),
    ],
)


def _split_edges(ei_row):
    r = ei_row.reshape(NCH, CHUNK)
    base = r[:NUM_SUBCORES * CH_PER_TILE].reshape(NUM_SUBCORES, CH_PER_TILE,
                                                  CHUNK)
    extra = jnp.concatenate(
        [r[NUM_SUBCORES * CH_PER_TILE:],
         jnp.zeros((NUM_SUBCORES - CH_EXTRA, CHUNK), jnp.int32)], axis=0)
    return jnp.concatenate([base, extra[:, None, :]], axis=1)


def kernel(A_x, B_x, A_edge_index, B_edge_index,
           Wl1, bl1, Wr1, Wl2, bl2, Wr2, Pw1, Pb1, Pw2, Pb2):
    xsplit = jnp.stack([jnp.stack([A_x[:, :HALF], A_x[:, HALF:]]),
                        jnp.stack([B_x[:, :HALF], B_x[:, HALF:]])])
    srcs = jnp.stack([_split_edges(A_edge_index[0]),
                      _split_edges(B_edge_index[0])])
    dsts = jnp.stack([_split_edges(A_edge_index[1]),
                      _split_edges(B_edge_index[1])])
    zrows = jnp.zeros((RPT, HALF), jnp.float32)
    zcnt = jnp.zeros((N_NODES,), jnp.float32)

    agg1, cnt = _make_segsum(True)(xsplit, srcs, dsts, zrows, zcnt)
    cnt3 = cnt[:, :, None]
    h1 = _dense1(agg1, cnt3, xsplit, Wl1.T, bl1[None], Wr1.T)
    (agg2,) = _make_segsum(False)(h1, srcs, dsts, zrows, zcnt)
    h, z = _dense2(agg2, cnt3, h1, Wl2.T, bl2[None], Wr2.T,
                   Pw1.T, Pb1[None], Pw2.T, Pb2[None])
    return (h[0], h[1], z[0], z[1])


# speedup vs baseline: 8.0040x; 1.0039x over previous
import functools

import jax
import jax.numpy as jnp
from jax import lax
from jax.experimental import pallas as pl
from jax.experimental.pallas import tpu as pltpu
from jax.experimental.pallas import tpu_sc as plsc

N_NODES = 10000
N_EDGES = 320000
DIM = 128
HALF = DIM // 2
PROJ = 64
NUM_CORES = 2
NUM_SUBCORES = 16
CHUNK = 128
NCH = N_EDGES // CHUNK
CH_PER_TILE = NCH // NUM_SUBCORES
CH_EXTRA = NCH - CH_PER_TILE * NUM_SUBCORES
CH_BUF = CH_PER_TILE + 1
RPT = 632
RPT_LAST = N_NODES - (NUM_SUBCORES - 1) * RPT
NBUF = 6
AHEAD = 3
ROW_BLK = 2000


def _make_segsum(layer1: bool):
    mesh = plsc.VectorSubcoreMesh(core_axis_name="c", subcore_axis_name="s",
                                  num_cores=NUM_CORES, num_subcores=NUM_SUBCORES)
    with_cnt = layer1
    out_type = [jax.ShapeDtypeStruct((NUM_CORES, 2, N_NODES, HALF),
                                     jnp.float32)]
    if with_cnt:
        out_type.append(jax.ShapeDtypeStruct((NUM_CORES, N_NODES), jnp.float32))
    scratch = (
        [pltpu.VMEM((CH_BUF, CHUNK), jnp.int32)] * 2
        + [pltpu.VMEM((CHUNK, HALF), jnp.float32)] * NBUF
        + [pltpu.VMEM((CHUNK,), jnp.float32),
           pltpu.VMEM_SHARED((N_NODES, HALF), jnp.float32),
           pltpu.VMEM_SHARED((N_NODES,), jnp.float32)]
        + [pltpu.SemaphoreType.DMA] * (3 * NBUF)
    )

    def body(x_hbm, ei_a_hbm, ei_b_hbm, zrows_hbm, zcnt_hbm, agg_hbm, *rest):
        if with_cnt:
            cnt_hbm = rest[0]
            rest = rest[1:]
        src_v, dst_v = rest[0], rest[1]
        rest = rest[2:]
        rows = rest[:NBUF]
        ones_v, acc_sh, cnt_sh = rest[NBUF:3 + NBUF]
        sems = rest[3 + NBUF:]
        gsem, ssem, csem = (sems[:NBUF], sems[NBUF:2 * NBUF],
                            sems[2 * NBUF:3 * NBUF])
        core = lax.axis_index("c")
        sub = lax.axis_index("s")

        def zero_acc():
            @pl.when(sub < NUM_SUBCORES - 1)
            def _zero_full():
                pltpu.sync_copy(zrows_hbm, acc_sh.at[pl.ds(sub * RPT, RPT)])

            @pl.when(sub == NUM_SUBCORES - 1)
            def _zero_last():
                pltpu.sync_copy(
                    zrows_hbm.at[pl.ds(0, RPT_LAST)],
                    acc_sh.at[pl.ds((NUM_SUBCORES - 1) * RPT, RPT_LAST)])

        def write_acc(half):
            @pl.when(sub < NUM_SUBCORES - 1)
            def _out_full():
                pltpu.sync_copy(acc_sh.at[pl.ds(sub * RPT, RPT)],
                                agg_hbm.at[core].at[half]
                                .at[pl.ds(sub * RPT, RPT)])

            @pl.when(sub == NUM_SUBCORES - 1)
            def _out_last():
                pltpu.sync_copy(
                    acc_sh.at[pl.ds((NUM_SUBCORES - 1) * RPT, RPT_LAST)],
                    agg_hbm.at[core].at[half]
                    .at[pl.ds((NUM_SUBCORES - 1) * RPT, RPT_LAST)])

        if with_cnt:
            @pl.when(sub == 0)
            def _zero_cnt():
                pltpu.sync_copy(zcnt_hbm, cnt_sh)
            for i in range(CHUNK // 16):
                ones_v[pl.ds(i * 16, 16)] = jnp.ones((16,), jnp.float32)

        def stage_row(b, buf):
            def one(ei_hbm):
                pltpu.sync_copy(
                    ei_hbm.at[b].at[pl.ds(sub * CH_PER_TILE, CH_PER_TILE)],
                    buf.at[pl.ds(0, CH_PER_TILE)])

                @pl.when(sub < CH_EXTRA)
                def _extra():
                    pltpu.sync_copy(
                        ei_hbm.at[b].at[pl.ds(NCH - CH_EXTRA + sub, 1)],
                        buf.at[pl.ds(CH_PER_TILE, 1)])

            @pl.when(core == 0)
            def _stage_a():
                one(ei_a_hbm)

            @pl.when(core == 1)
            def _stage_b():
                one(ei_b_hbm)

        stage_row(0, src_v)
        stage_row(2 if layer1 else 1, dst_v)

        nch = CH_PER_TILE + jnp.where(sub < CH_EXTRA, 1, 0)
        zero_acc()
        plsc.subcore_barrier()

        nquad = CH_PER_TILE // NBUF

        for half in range(2):
            cnt_here = with_cnt and half == 0
            if layer1:
                table = x_hbm.at[core]
                if half == 1:
                    stage_row(1, src_v)
            else:
                table = x_hbm.at[core].at[half]

            def g_start(j, k):
                pltpu.async_copy(table.at[src_v.at[j]], rows[k], gsem[k])

            def g_wait(k):
                pltpu.make_async_copy(table.at[src_v.at[0]], rows[k],
                                      gsem[k]).wait()

            def s_start(j, k):
                pltpu.async_copy(rows[k], acc_sh.at[dst_v.at[j]], ssem[k],
                                 add=True)

            def s_wait(k):
                pltpu.make_async_copy(rows[k], acc_sh.at[dst_v.at[0]],
                                      ssem[k]).wait()

            def c_start(j, k):
                pltpu.async_copy(ones_v, cnt_sh.at[dst_v.at[j]], csem[k],
                                 add=True)

            def c_wait(k):
                pltpu.make_async_copy(ones_v, cnt_sh.at[dst_v.at[0]],
                                      csem[k]).wait()

            for j0 in range(AHEAD):
                g_start(j0, j0)

            def ring(i, carry):
                for k in range(NBUF):
                    j = NBUF * i + k
                    kd = (k + AHEAD) % NBUF

                    @pl.when(j >= AHEAD)
                    def _drain():
                        s_wait(kd)
                        if cnt_here:
                            c_wait(kd)

                    @pl.when(j + AHEAD < nch)
                    def _prefetch():
                        g_start(j + AHEAD, kd)

                    g_wait(k)
                    s_start(j, k)
                    if cnt_here:
                        c_start(j, k)
                return carry

            lax.fori_loop(0, nquad, ring, 0)
            for k in range(NBUF - AHEAD, NBUF):
                s_wait(k)
                if cnt_here:
                    c_wait(k)

            @pl.when(nch > CH_PER_TILE)
            def _extra_chunk():
                g_wait(CH_PER_TILE % NBUF)
                pltpu.sync_copy(rows[CH_PER_TILE % NBUF],
                                acc_sh.at[dst_v.at[CH_PER_TILE]], add=True)
                if cnt_here:
                    pltpu.sync_copy(ones_v, cnt_sh.at[dst_v.at[CH_PER_TILE]],
                                    add=True)

            plsc.subcore_barrier()
            write_acc(half)
            if half == 0:
                if with_cnt:
                    @pl.when(sub == 0)
                    def _out_cnt():
                        pltpu.sync_copy(cnt_sh, cnt_hbm.at[core])
                zero_acc()
                plsc.subcore_barrier()

    return pl.kernel(body, out_type=tuple(out_type), mesh=mesh,
                     scratch_types=scratch,
                     compiler_params=pltpu.CompilerParams(
                         use_tc_tiling_on_sc=False))


_make_segsum = functools.lru_cache(maxsize=None)(_make_segsum)


def _agg_term(agg_ref, cnt_ref, wl_ref, bl_ref):
    inv = 1.0 / jnp.maximum(cnt_ref[0], 1.0)
    return (jnp.dot(agg_ref[0, 0] * inv, wl_ref[:HALF],
                    preferred_element_type=jnp.float32)
            + jnp.dot(agg_ref[0, 1] * inv, wl_ref[HALF:],
                      preferred_element_type=jnp.float32)
            + bl_ref[...])


def _dense1_body(agg_ref, cnt_ref, x_ref, wl_ref, bl_ref, wr_ref, h_ref):
    h = (_agg_term(agg_ref, cnt_ref, wl_ref, bl_ref)
         + jnp.dot(x_ref[0], wr_ref[...], preferred_element_type=jnp.float32))
    h = jnp.maximum(h, 0.0)
    h_ref[0, 0] = h[:, :HALF]
    h_ref[0, 1] = h[:, HALF:]


def _dense2_body(agg_ref, cnt_ref, x_ref, wl_ref, bl_ref, wr_ref,
                 pw1_ref, pb1_ref, pw2_ref, pb2_ref, h_ref, z_ref):
    h = (_agg_term(agg_ref, cnt_ref, wl_ref, bl_ref)
         + jnp.dot(x_ref[0, 0], wr_ref[:HALF],
                   preferred_element_type=jnp.float32)
         + jnp.dot(x_ref[0, 1], wr_ref[HALF:],
                   preferred_element_type=jnp.float32))
    h_ref[0] = h
    t = jnp.maximum(
        jnp.dot(h, pw1_ref[...], preferred_element_type=jnp.float32)
        + pb1_ref[...], 0.0)
    z = (jnp.dot(t, pw2_ref[...], preferred_element_type=jnp.float32)
         + pb2_ref[...])
    nrm = jnp.sqrt(jnp.sum(z * z, axis=1, keepdims=True))
    z_ref[0] = z / jnp.maximum(nrm, 1e-12)


def _row_blocks(feat):
    return pl.BlockSpec((1, ROW_BLK, feat), lambda b, r: (b, r, 0))


def _split_blocks():
    return pl.BlockSpec((1, 2, ROW_BLK, HALF), lambda b, r: (b, 0, r, 0))


def _whole(shape):
    return pl.BlockSpec(shape, lambda b, r: tuple(0 for _ in shape))


_GRID = (2, N_NODES // ROW_BLK)

_SPLIT_SHAPE = jax.ShapeDtypeStruct((2, 2, N_NODES, HALF), jnp.float32)

_dense1 = pl.pallas_call(
    _dense1_body,
    grid=_GRID,
    in_specs=[
        _split_blocks(),
        _row_blocks(1),
        _row_blocks(DIM),
        _whole((DIM, DIM)),
        _whole((1, DIM)),
        _whole((DIM, DIM)),
    ],
    out_specs=_split_blocks(),
    out_shape=_SPLIT_SHAPE,
)

_dense2 = pl.pallas_call(
    _dense2_body,
    grid=_GRID,
    in_specs=[
        _split_blocks(),
        _row_blocks(1),
        _split_blocks(),
        _whole((DIM, DIM)),
        _whole((1, DIM)),
        _whole((DIM, DIM)),
        _whole((DIM, PROJ)),
        _whole((1, PROJ)),
        _whole((PROJ, PROJ)),
        _whole((1, PROJ)),
    ],
    out_specs=[_row_blocks(DIM), _row_blocks(PROJ)],
    out_shape=[
        jax.ShapeDtypeStruct((2, N_NODES, DIM), jnp.float32),
        jax.ShapeDtypeStruct((2, N_NODES, PROJ), jnp.float32),
    ],
)


def _edges_l1(ei):
    s2 = ei[0] * 2
    return jnp.stack([s2, s2 + 1, ei[1]]).reshape(3, NCH, CHUNK)


def kernel(A_x, B_x, A_edge_index, B_edge_index,
           Wl1, bl1, Wr1, Wl2, bl2, Wr2, Pw1, Pb1, Pw2, Pb2):
    xs = jnp.stack([A_x, B_x])
    xs2 = xs.reshape(2, 2 * N_NODES, HALF)
    ea1, eb1 = _edges_l1(A_edge_index), _edges_l1(B_edge_index)
    ea2 = A_edge_index.reshape(2, NCH, CHUNK)
    eb2 = B_edge_index.reshape(2, NCH, CHUNK)
    zrows = jnp.zeros((RPT, HALF), jnp.float32)
    zcnt = jnp.zeros((N_NODES,), jnp.float32)

    agg1, cnt = _make_segsum(True)(xs2, ea1, eb1, zrows, zcnt)
    cnt3 = cnt[:, :, None]
    h1 = _dense1(agg1, cnt3, xs, Wl1.T, bl1[None], Wr1.T)
    (agg2,) = _make_segsum(False)(h1, ea2, eb2, zrows, zcnt)
    h, z = _dense2(agg2, cnt3, h1, Wl2.T, bl2[None], Wr2.T,
                   Pw1.T, Pb1[None], Pw2.T, Pb2[None])
    return (h[0], h[1], z[0], z[1])


# speedup vs baseline: 8.0340x; 1.0037x over previous
import functools

import jax
import jax.numpy as jnp
from jax import lax
from jax.experimental import pallas as pl
from jax.experimental.pallas import tpu as pltpu
from jax.experimental.pallas import tpu_sc as plsc

N_NODES = 10000
N_EDGES = 320000
DIM = 128
HALF = DIM // 2
PROJ = 64
NUM_CORES = 2
NUM_SUBCORES = 16
CHUNK = 128
NCH = N_EDGES // CHUNK
CH_PER_TILE = NCH // NUM_SUBCORES
CH_EXTRA = NCH - CH_PER_TILE * NUM_SUBCORES
CH_BUF = CH_PER_TILE + 1
RPT = 632
RPT_LAST = N_NODES - (NUM_SUBCORES - 1) * RPT
NBUF = 6
AHEAD = 3
ROW_BLK = 2000


def _make_segsum(layer1: bool):
    mesh = plsc.VectorSubcoreMesh(core_axis_name="c", subcore_axis_name="s",
                                  num_cores=NUM_CORES, num_subcores=NUM_SUBCORES)
    with_cnt = layer1
    out_type = [jax.ShapeDtypeStruct((NUM_CORES, 2, N_NODES, HALF),
                                     jnp.float32)]
    if with_cnt:
        out_type.append(jax.ShapeDtypeStruct((NUM_CORES, N_NODES), jnp.float32))
    scratch = (
        [pltpu.VMEM((CH_BUF, CHUNK), jnp.int32)] * 2
        + [pltpu.VMEM((CHUNK, HALF), jnp.float32)] * NBUF
        + [pltpu.VMEM((CHUNK,), jnp.float32),
           pltpu.VMEM_SHARED((N_NODES, HALF), jnp.float32),
           pltpu.VMEM_SHARED((N_NODES,), jnp.float32)]
        + [pltpu.SemaphoreType.DMA] * (3 * NBUF)
    )

    def body(x_hbm, ei_a_hbm, ei_b_hbm, zrows_hbm, zcnt_hbm, agg_hbm, *rest):
        if with_cnt:
            cnt_hbm = rest[0]
            rest = rest[1:]
        src_v, dst_v = rest[0], rest[1]
        rest = rest[2:]
        rows = rest[:NBUF]
        ones_v, acc_sh, cnt_sh = rest[NBUF:3 + NBUF]
        sems = rest[3 + NBUF:]
        gsem, ssem, csem = (sems[:NBUF], sems[NBUF:2 * NBUF],
                            sems[2 * NBUF:3 * NBUF])
        core = lax.axis_index("c")
        sub = lax.axis_index("s")

        def zero_acc():
            @pl.when(sub < NUM_SUBCORES - 1)
            def _zero_full():
                pltpu.sync_copy(zrows_hbm, acc_sh.at[pl.ds(sub * RPT, RPT)])

            @pl.when(sub == NUM_SUBCORES - 1)
            def _zero_last():
                pltpu.sync_copy(
                    zrows_hbm.at[pl.ds(0, RPT_LAST)],
                    acc_sh.at[pl.ds((NUM_SUBCORES - 1) * RPT, RPT_LAST)])

        def write_acc(half):
            @pl.when(sub < NUM_SUBCORES - 1)
            def _out_full():
                pltpu.sync_copy(acc_sh.at[pl.ds(sub * RPT, RPT)],
                                agg_hbm.at[core].at[half]
                                .at[pl.ds(sub * RPT, RPT)])

            @pl.when(sub == NUM_SUBCORES - 1)
            def _out_last():
                pltpu.sync_copy(
                    acc_sh.at[pl.ds((NUM_SUBCORES - 1) * RPT, RPT_LAST)],
                    agg_hbm.at[core].at[half]
                    .at[pl.ds((NUM_SUBCORES - 1) * RPT, RPT_LAST)])

        if with_cnt:
            @pl.when(sub == 0)
            def _zero_cnt():
                pltpu.sync_copy(zcnt_hbm, cnt_sh)
            for i in range(CHUNK // 16):
                ones_v[pl.ds(i * 16, 16)] = jnp.ones((16,), jnp.float32)

        def stage_row(b, buf):
            def one(ei_hbm):
                pltpu.sync_copy(
                    ei_hbm.at[b].at[pl.ds(sub * CH_PER_TILE, CH_PER_TILE)],
                    buf.at[pl.ds(0, CH_PER_TILE)])

                @pl.when(sub < CH_EXTRA)
                def _extra():
                    pltpu.sync_copy(
                        ei_hbm.at[b].at[pl.ds(NCH - CH_EXTRA + sub, 1)],
                        buf.at[pl.ds(CH_PER_TILE, 1)])

            @pl.when(core == 0)
            def _stage_a():
                one(ei_a_hbm)

            @pl.when(core == 1)
            def _stage_b():
                one(ei_b_hbm)

        stage_row(0 if layer1 else 3, src_v)
        stage_row(2, dst_v)

        nch = CH_PER_TILE + jnp.where(sub < CH_EXTRA, 1, 0)
        zero_acc()
        plsc.subcore_barrier()

        nquad = CH_PER_TILE // NBUF

        for half in range(2):
            cnt_here = with_cnt and half == 0
            if layer1:
                table = x_hbm.at[core]
                if half == 1:
                    stage_row(1, src_v)
            else:
                table = x_hbm.at[core].at[half]

            def g_start(j, k):
                pltpu.async_copy(table.at[src_v.at[j]], rows[k], gsem[k])

            def g_wait(k):
                pltpu.make_async_copy(table.at[src_v.at[0]], rows[k],
                                      gsem[k]).wait()

            def s_start(j, k):
                pltpu.async_copy(rows[k], acc_sh.at[dst_v.at[j]], ssem[k],
                                 add=True)

            def s_wait(k):
                pltpu.make_async_copy(rows[k], acc_sh.at[dst_v.at[0]],
                                      ssem[k]).wait()

            def c_start(j, k):
                pltpu.async_copy(ones_v, cnt_sh.at[dst_v.at[j]], csem[k],
                                 add=True)

            def c_wait(k):
                pltpu.make_async_copy(ones_v, cnt_sh.at[dst_v.at[0]],
                                      csem[k]).wait()

            for j0 in range(AHEAD):
                g_start(j0, j0)

            def ring(i, carry):
                for k in range(NBUF):
                    j = NBUF * i + k
                    kd = (k + AHEAD) % NBUF

                    @pl.when(j >= AHEAD)
                    def _drain():
                        s_wait(kd)
                        if cnt_here:
                            c_wait(kd)

                    @pl.when(j + AHEAD < nch)
                    def _prefetch():
                        g_start(j + AHEAD, kd)

                    g_wait(k)
                    s_start(j, k)
                    if cnt_here:
                        c_start(j, k)
                return carry

            lax.fori_loop(0, nquad, ring, 0)
            for k in range(NBUF - AHEAD, NBUF):
                s_wait(k)
                if cnt_here:
                    c_wait(k)

            @pl.when(nch > CH_PER_TILE)
            def _extra_chunk():
                g_wait(CH_PER_TILE % NBUF)
                pltpu.sync_copy(rows[CH_PER_TILE % NBUF],
                                acc_sh.at[dst_v.at[CH_PER_TILE]], add=True)
                if cnt_here:
                    pltpu.sync_copy(ones_v, cnt_sh.at[dst_v.at[CH_PER_TILE]],
                                    add=True)

            plsc.subcore_barrier()
            write_acc(half)
            if half == 0:
                if with_cnt:
                    @pl.when(sub == 0)
                    def _out_cnt():
                        pltpu.sync_copy(cnt_sh, cnt_hbm.at[core])
                zero_acc()
                plsc.subcore_barrier()

    return pl.kernel(body, out_type=tuple(out_type), mesh=mesh,
                     scratch_types=scratch,
                     compiler_params=pltpu.CompilerParams(
                         use_tc_tiling_on_sc=False))


_make_segsum = functools.lru_cache(maxsize=None)(_make_segsum)


def _agg_term(agg_ref, cnt_ref, wl_ref, bl_ref):
    inv = 1.0 / jnp.maximum(cnt_ref[0], 1.0)
    return (jnp.dot(agg_ref[0, 0] * inv, wl_ref[:HALF],
                    preferred_element_type=jnp.float32)
            + jnp.dot(agg_ref[0, 1] * inv, wl_ref[HALF:],
                      preferred_element_type=jnp.float32)
            + bl_ref[...])


def _dense1_body(agg_ref, cnt_ref, x_ref, wl_ref, bl_ref, wr_ref, h_ref):
    h = (_agg_term(agg_ref, cnt_ref, wl_ref, bl_ref)
         + jnp.dot(x_ref[0], wr_ref[...], preferred_element_type=jnp.float32))
    h = jnp.maximum(h, 0.0)
    h_ref[0, 0] = h[:, :HALF]
    h_ref[0, 1] = h[:, HALF:]


def _dense2_body(agg_ref, cnt_ref, x_ref, wl_ref, bl_ref, wr_ref,
                 pw1_ref, pb1_ref, pw2_ref, pb2_ref, h_ref, z_ref):
    h = (_agg_term(agg_ref, cnt_ref, wl_ref, bl_ref)
         + jnp.dot(x_ref[0, 0], wr_ref[:HALF],
                   preferred_element_type=jnp.float32)
         + jnp.dot(x_ref[0, 1], wr_ref[HALF:],
                   preferred_element_type=jnp.float32))
    h_ref[0] = h
    t = jnp.maximum(
        jnp.dot(h, pw1_ref[...], preferred_element_type=jnp.float32)
        + pb1_ref[...], 0.0)
    z = (jnp.dot(t, pw2_ref[...], preferred_element_type=jnp.float32)
         + pb2_ref[...])
    nrm = jnp.sqrt(jnp.sum(z * z, axis=1, keepdims=True))
    z_ref[0] = z / jnp.maximum(nrm, 1e-12)


def _row_blocks(feat):
    return pl.BlockSpec((1, ROW_BLK, feat), lambda b, r: (b, r, 0))


def _split_blocks():
    return pl.BlockSpec((1, 2, ROW_BLK, HALF), lambda b, r: (b, 0, r, 0))


def _whole(shape):
    return pl.BlockSpec(shape, lambda b, r: tuple(0 for _ in shape))


_GRID = (2, N_NODES // ROW_BLK)

_SPLIT_SHAPE = jax.ShapeDtypeStruct((2, 2, N_NODES, HALF), jnp.float32)

_dense1 = pl.pallas_call(
    _dense1_body,
    grid=_GRID,
    in_specs=[
        _split_blocks(),
        _row_blocks(1),
        _row_blocks(DIM),
        _whole((DIM, DIM)),
        _whole((1, DIM)),
        _whole((DIM, DIM)),
    ],
    out_specs=_split_blocks(),
    out_shape=_SPLIT_SHAPE,
)

_dense2 = pl.pallas_call(
    _dense2_body,
    grid=_GRID,
    in_specs=[
        _split_blocks(),
        _row_blocks(1),
        _split_blocks(),
        _whole((DIM, DIM)),
        _whole((1, DIM)),
        _whole((DIM, DIM)),
        _whole((DIM, PROJ)),
        _whole((1, PROJ)),
        _whole((PROJ, PROJ)),
        _whole((1, PROJ)),
    ],
    out_specs=[_row_blocks(DIM), _row_blocks(PROJ)],
    out_shape=[
        jax.ShapeDtypeStruct((2, N_NODES, DIM), jnp.float32),
        jax.ShapeDtypeStruct((2, N_NODES, PROJ), jnp.float32),
    ],
)


def _edge_rows(ei):
    s, d = ei[0], ei[1]
    return jnp.concatenate([s * 2, s * 2 + 1, d, s]).reshape(4, NCH, CHUNK)


def kernel(A_x, B_x, A_edge_index, B_edge_index,
           Wl1, bl1, Wr1, Wl2, bl2, Wr2, Pw1, Pb1, Pw2, Pb2):
    xs = jnp.stack([A_x, B_x])
    xs2 = jnp.concatenate([A_x.reshape(-1), B_x.reshape(-1)]).reshape(
        2, 2 * N_NODES, HALF)
    ea, eb = _edge_rows(A_edge_index), _edge_rows(B_edge_index)
    zrows = jnp.zeros((RPT, HALF), jnp.float32)
    zcnt = jnp.zeros((N_NODES,), jnp.float32)

    agg1, cnt = _make_segsum(True)(xs2, ea, eb, zrows, zcnt)
    cnt3 = cnt[:, :, None]
    h1 = _dense1(agg1, cnt3, xs, Wl1.T, bl1[None], Wr1.T)
    (agg2,) = _make_segsum(False)(h1, ea, eb, zrows, zcnt)
    h, z = _dense2(agg2, cnt3, h1, Wl2.T, bl2[None], Wr2.T,
                   Pw1.T, Pb1[None], Pw2.T, Pb2[None])
    return (h[0], h[1], z[0], z[1])


# speedup vs baseline: 8.6668x; 1.0788x over previous
import functools

import jax
import jax.numpy as jnp
from jax import lax
from jax.experimental import pallas as pl
from jax.experimental.pallas import tpu as pltpu
from jax.experimental.pallas import tpu_sc as plsc

N_NODES = 10000
N_EDGES = 320000
DIM = 128
HALF = DIM // 2
PROJ = 64
NUM_CORES = 2
NUM_SUBCORES = 16
CHUNK = 128
NCH = N_EDGES // CHUNK
CH_PER_TILE = NCH // NUM_SUBCORES
CH_EXTRA = NCH - CH_PER_TILE * NUM_SUBCORES
CH_BUF = CH_PER_TILE + 1
RPT = 632
RPT_LAST = N_NODES - (NUM_SUBCORES - 1) * RPT
NBUF = 6
AHEAD = 3
ROW_BLK = 2000


def _make_segsum(layer1: bool):
    mesh = plsc.VectorSubcoreMesh(core_axis_name="c", subcore_axis_name="s",
                                  num_cores=NUM_CORES, num_subcores=NUM_SUBCORES)
    with_cnt = layer1
    out_type = [jax.ShapeDtypeStruct((NUM_CORES, 2, N_NODES, HALF),
                                     jnp.float32)]
    if with_cnt:
        out_type.append(jax.ShapeDtypeStruct((NUM_CORES, N_NODES), jnp.float32))
    scratch = (
        [pltpu.VMEM((CH_BUF, CHUNK), jnp.int32)] * 2
        + [pltpu.VMEM((CHUNK, HALF), jnp.float32)] * NBUF
        + [pltpu.VMEM((CHUNK,), jnp.float32),
           pltpu.VMEM_SHARED((N_NODES, HALF), jnp.float32),
           pltpu.VMEM_SHARED((N_NODES,), jnp.float32)]
        + [pltpu.SemaphoreType.DMA] * (3 * NBUF)
    )

    def body(*args):
        if layer1:
            xa_hbm, xb_hbm = args[0], args[1]
            args = args[2:]
        else:
            x_hbm = args[0]
            args = args[1:]
        ei_a_hbm, ei_b_hbm, zrows_hbm, zcnt_hbm, agg_hbm = args[:5]
        rest = args[5:]
        if with_cnt:
            cnt_hbm = rest[0]
            rest = rest[1:]
        src_v, dst_v = rest[0], rest[1]
        rest = rest[2:]
        rows = rest[:NBUF]
        ones_v, acc_sh, cnt_sh = rest[NBUF:3 + NBUF]
        sems = rest[3 + NBUF:]
        gsem, ssem, csem = (sems[:NBUF], sems[NBUF:2 * NBUF],
                            sems[2 * NBUF:3 * NBUF])
        core = lax.axis_index("c")
        sub = lax.axis_index("s")

        def zero_acc():
            @pl.when(sub < NUM_SUBCORES - 1)
            def _zero_full():
                pltpu.sync_copy(zrows_hbm, acc_sh.at[pl.ds(sub * RPT, RPT)])

            @pl.when(sub == NUM_SUBCORES - 1)
            def _zero_last():
                pltpu.sync_copy(
                    zrows_hbm.at[pl.ds(0, RPT_LAST)],
                    acc_sh.at[pl.ds((NUM_SUBCORES - 1) * RPT, RPT_LAST)])

        def write_acc(half):
            @pl.when(sub < NUM_SUBCORES - 1)
            def _out_full():
                pltpu.sync_copy(acc_sh.at[pl.ds(sub * RPT, RPT)],
                                agg_hbm.at[core].at[half]
                                .at[pl.ds(sub * RPT, RPT)])

            @pl.when(sub == NUM_SUBCORES - 1)
            def _out_last():
                pltpu.sync_copy(
                    acc_sh.at[pl.ds((NUM_SUBCORES - 1) * RPT, RPT_LAST)],
                    agg_hbm.at[core].at[half]
                    .at[pl.ds((NUM_SUBCORES - 1) * RPT, RPT_LAST)])

        if with_cnt:
            @pl.when(sub == 0)
            def _zero_cnt():
                pltpu.sync_copy(zcnt_hbm, cnt_sh)
            for i in range(CHUNK // 16):
                ones_v[pl.ds(i * 16, 16)] = jnp.ones((16,), jnp.float32)

        def stage_row(b, buf):
            def one(ei_hbm):
                pltpu.sync_copy(
                    ei_hbm.at[b].at[pl.ds(sub * CH_PER_TILE, CH_PER_TILE)],
                    buf.at[pl.ds(0, CH_PER_TILE)])

                @pl.when(sub < CH_EXTRA)
                def _extra():
                    pltpu.sync_copy(
                        ei_hbm.at[b].at[pl.ds(NCH - CH_EXTRA + sub, 1)],
                        buf.at[pl.ds(CH_PER_TILE, 1)])

            @pl.when(core == 0)
            def _stage_a():
                one(ei_a_hbm)

            @pl.when(core == 1)
            def _stage_b():
                one(ei_b_hbm)

        stage_row(0 if layer1 else 3, src_v)
        stage_row(2, dst_v)

        nch = CH_PER_TILE + jnp.where(sub < CH_EXTRA, 1, 0)
        zero_acc()
        plsc.subcore_barrier()

        nquad = CH_PER_TILE // NBUF

        def run_ring(table, cnt_here):
            def g_start(j, k):
                pltpu.async_copy(table.at[src_v.at[j]], rows[k], gsem[k])

            def g_wait(k):
                pltpu.make_async_copy(table.at[src_v.at[0]], rows[k],
                                      gsem[k]).wait()

            def s_start(j, k):
                pltpu.async_copy(rows[k], acc_sh.at[dst_v.at[j]], ssem[k],
                                 add=True)

            def s_wait(k):
                pltpu.make_async_copy(rows[k], acc_sh.at[dst_v.at[0]],
                                      ssem[k]).wait()

            def c_start(j, k):
                pltpu.async_copy(ones_v, cnt_sh.at[dst_v.at[j]], csem[k],
                                 add=True)

            def c_wait(k):
                pltpu.make_async_copy(ones_v, cnt_sh.at[dst_v.at[0]],
                                      csem[k]).wait()

            for j0 in range(AHEAD):
                g_start(j0, j0)

            def ring(i, carry):
                for k in range(NBUF):
                    j = NBUF * i + k
                    kd = (k + AHEAD) % NBUF

                    @pl.when(j >= AHEAD)
                    def _drain():
                        s_wait(kd)
                        if cnt_here:
                            c_wait(kd)

                    @pl.when(j + AHEAD < nch)
                    def _prefetch():
                        g_start(j + AHEAD, kd)

                    g_wait(k)
                    s_start(j, k)
                    if cnt_here:
                        c_start(j, k)
                return carry

            lax.fori_loop(0, nquad, ring, 0)
            for k in range(NBUF - AHEAD, NBUF):
                s_wait(k)
                if cnt_here:
                    c_wait(k)

            @pl.when(nch > CH_PER_TILE)
            def _extra_chunk():
                g_wait(CH_PER_TILE % NBUF)
                pltpu.sync_copy(rows[CH_PER_TILE % NBUF],
                                acc_sh.at[dst_v.at[CH_PER_TILE]], add=True)
                if cnt_here:
                    pltpu.sync_copy(ones_v, cnt_sh.at[dst_v.at[CH_PER_TILE]],
                                    add=True)

        for half in range(2):
            cnt_here = with_cnt and half == 0
            if layer1:
                if half == 1:
                    stage_row(1, src_v)

                @pl.when(core == 0)
                def _ring_a():
                    run_ring(xa_hbm, cnt_here)

                @pl.when(core == 1)
                def _ring_b():
                    run_ring(xb_hbm, cnt_here)
            else:
                run_ring(x_hbm.at[core].at[half], cnt_here)

            plsc.subcore_barrier()
            write_acc(half)
            if half == 0:
                if with_cnt:
                    @pl.when(sub == 0)
                    def _out_cnt():
                        pltpu.sync_copy(cnt_sh, cnt_hbm.at[core])
                zero_acc()
                plsc.subcore_barrier()

    return pl.kernel(body, out_type=tuple(out_type), mesh=mesh,
                     scratch_types=scratch,
                     compiler_params=pltpu.CompilerParams(
                         use_tc_tiling_on_sc=False))


_make_segsum = functools.lru_cache(maxsize=None)(_make_segsum)


def _agg_term(agg_ref, cnt_ref, wl_ref, bl_ref):
    inv = 1.0 / jnp.maximum(cnt_ref[0], 1.0)
    return (jnp.dot(agg_ref[0, 0] * inv, wl_ref[:HALF],
                    preferred_element_type=jnp.float32)
            + jnp.dot(agg_ref[0, 1] * inv, wl_ref[HALF:],
                      preferred_element_type=jnp.float32)
            + bl_ref[...])


def _dense1_body(agg_ref, cnt_ref, x_ref, wl_ref, bl_ref, wr_ref, h_ref):
    h = (_agg_term(agg_ref, cnt_ref, wl_ref, bl_ref)
         + jnp.dot(x_ref[0], wr_ref[...], preferred_element_type=jnp.float32))
    h = jnp.maximum(h, 0.0)
    h_ref[0, 0] = h[:, :HALF]
    h_ref[0, 1] = h[:, HALF:]


def _dense2_body(agg_ref, cnt_ref, x_ref, wl_ref, bl_ref, wr_ref,
                 pw1_ref, pb1_ref, pw2_ref, pb2_ref, h_ref, z_ref):
    h = (_agg_term(agg_ref, cnt_ref, wl_ref, bl_ref)
         + jnp.dot(x_ref[0, 0], wr_ref[:HALF],
                   preferred_element_type=jnp.float32)
         + jnp.dot(x_ref[0, 1], wr_ref[HALF:],
                   preferred_element_type=jnp.float32))
    h_ref[0] = h
    t = jnp.maximum(
        jnp.dot(h, pw1_ref[...], preferred_element_type=jnp.float32)
        + pb1_ref[...], 0.0)
    z = (jnp.dot(t, pw2_ref[...], preferred_element_type=jnp.float32)
         + pb2_ref[...])
    nrm = jnp.sqrt(jnp.sum(z * z, axis=1, keepdims=True))
    z_ref[0] = z / jnp.maximum(nrm, 1e-12)


def _row_blocks(feat):
    return pl.BlockSpec((1, ROW_BLK, feat), lambda b, r: (b, r, 0))


def _split_blocks():
    return pl.BlockSpec((1, 2, ROW_BLK, HALF), lambda b, r: (b, 0, r, 0))


def _whole(shape):
    return pl.BlockSpec(shape, lambda b, r: tuple(0 for _ in shape))


_GRID = (2, N_NODES // ROW_BLK)

_SPLIT_SHAPE = jax.ShapeDtypeStruct((2, 2, N_NODES, HALF), jnp.float32)

_dense1 = pl.pallas_call(
    _dense1_body,
    grid=_GRID,
    in_specs=[
        _split_blocks(),
        _row_blocks(1),
        _row_blocks(DIM),
        _whole((DIM, DIM)),
        _whole((1, DIM)),
        _whole((DIM, DIM)),
    ],
    out_specs=_split_blocks(),
    out_shape=_SPLIT_SHAPE,
)

_dense2 = pl.pallas_call(
    _dense2_body,
    grid=_GRID,
    in_specs=[
        _split_blocks(),
        _row_blocks(1),
        _split_blocks(),
        _whole((DIM, DIM)),
        _whole((1, DIM)),
        _whole((DIM, DIM)),
        _whole((DIM, PROJ)),
        _whole((1, PROJ)),
        _whole((PROJ, PROJ)),
        _whole((1, PROJ)),
    ],
    out_specs=[_row_blocks(DIM), _row_blocks(PROJ)],
    out_shape=[
        jax.ShapeDtypeStruct((2, N_NODES, DIM), jnp.float32),
        jax.ShapeDtypeStruct((2, N_NODES, PROJ), jnp.float32),
    ],
)


def _edge_rows(ei):
    s, d = ei[0], ei[1]
    return jnp.concatenate([s * 2, s * 2 + 1, d, s]).reshape(4, NCH, CHUNK)


def kernel(A_x, B_x, A_edge_index, B_edge_index,
           Wl1, bl1, Wr1, Wl2, bl2, Wr2, Pw1, Pb1, Pw2, Pb2):
    xs = jnp.stack([A_x, B_x])
    xa2 = A_x.reshape(2 * N_NODES, HALF)
    xb2 = B_x.reshape(2 * N_NODES, HALF)
    ea, eb = _edge_rows(A_edge_index), _edge_rows(B_edge_index)
    zrows = jnp.zeros((RPT, HALF), jnp.float32)
    zcnt = jnp.zeros((N_NODES,), jnp.float32)

    agg1, cnt = _make_segsum(True)(xa2, xb2, ea, eb, zrows, zcnt)
    cnt3 = cnt[:, :, None]
    h1 = _dense1(agg1, cnt3, xs, Wl1.T, bl1[None], Wr1.T)
    (agg2,) = _make_segsum(False)(h1, ea, eb, zrows, zcnt)
    h, z = _dense2(agg2, cnt3, h1, Wl2.T, bl2[None], Wr2.T,
                   Pw1.T, Pb1[None], Pw2.T, Pb2[None])
    return (h[0], h[1], z[0], z[1])
